# baseline (speedup 1.0000x reference)
import functools

import jax
import jax.numpy as jnp
from jax import lax
from jax.experimental import pallas as pl
from jax.experimental.pallas import tpu as pltpu
from jax.experimental.pallas import tpu_sc as plsc

VOCAB = 1000000
D = 32
B = 16384
L = 200
NCLS = 4

NC, NS = 2, 16
NW = NC * NS
BAGS_PER_W = B // NW
NB = 16
NCHUNK = BAGS_PER_W // NB
IDX_ROWS = NB * L // 128


ROWS_STEP = 128
GROUPS = VOCAB // 16
GROUPS_PER_W = -(-GROUPS // NW)
NSTEPS = 246
LAST_ROW0 = VOCAB - ROWS_STEP


def _sc_project(emb, wtile):
    mesh = plsc.VectorSubcoreMesh(core_axis_name="c", subcore_axis_name="s")

    @functools.partial(
        pl.kernel,
        mesh=mesh,
        out_type=jax.ShapeDtypeStruct((VOCAB, 16), jnp.float32),
        scratch_types=[
            pltpu.VMEM((ROWS_STEP, D), jnp.float32),
            pltpu.VMEM((ROWS_STEP, D), jnp.float32),
            pltpu.VMEM((ROWS_STEP, 16), jnp.float32),
            pltpu.VMEM((ROWS_STEP, 16), jnp.float32),
            pltpu.VMEM((D, 16), jnp.float32),
            pltpu.SemaphoreType.DMA,
            pltpu.SemaphoreType.DMA,
            pltpu.SemaphoreType.DMA,
            pltpu.SemaphoreType.DMA,
        ],
        compiler_params=pltpu.CompilerParams(
            use_tc_tiling_on_sc=False, needs_layout_passes=False),
    )
    def k(emb_hbm, w_hbm, out_hbm, in0, in1, ob0, ob1, wv, isem0, isem1,
          osem0, osem1):
        wid = lax.axis_index("s") * NC + lax.axis_index("c")
        wbase = wid * (GROUPS_PER_W * 16)
        pltpu.sync_copy(w_hbm, wv)
        wvk = [wv[kk, :] for kk in range(D)]
        lane = lax.iota(jnp.int32, 16)
        div4 = lane // 4
        mod4 = lane % 4
        zero = jnp.zeros((16,), jnp.float32)

        @pl.loop(0, ROWS_STEP)
        def _(r):
            ob0[r, :] = zero
            ob1[r, :] = zero

        def row0_of(s):
            return jnp.minimum(wbase + s * ROWS_STEP, LAST_ROW0)

        def start_in(s, buf, sem):
            return pltpu.async_copy(
                emb_hbm.at[pl.ds(row0_of(s), ROWS_STEP)], buf, sem)

        def compute(buf, obuf):
            for r4 in range(ROWS_STEP // 4):
                rbase = r4 * 4 + div4
                acc = zero
                for kk in range(D):
                    col = plsc.load_gather(
                        buf, [rbase, jnp.full((16,), kk, jnp.int32)])
                    acc = acc + col * wvk[kk]
                plsc.store_scatter(obuf, [rbase, mod4], acc)

        start_in(0, in0, isem0)

        @pl.loop(0, NSTEPS // 2)
        def _(t):
            s0 = 2 * t
            start_in(s0 + 1, in1, isem1)
            pltpu.make_async_copy(
                emb_hbm.at[pl.ds(0, ROWS_STEP)], in0, isem0).wait()

            @pl.when(t > 0)
            def _():
                pltpu.make_async_copy(
                    ob0, out_hbm.at[pl.ds(0, ROWS_STEP)], osem0).wait()

            compute(in0, ob0)
            pltpu.async_copy(ob0, out_hbm.at[pl.ds(row0_of(s0), ROWS_STEP)],
                             osem0)
            start_in(s0 + 2, in0, isem0)
            pltpu.make_async_copy(
                emb_hbm.at[pl.ds(0, ROWS_STEP)], in1, isem1).wait()

            @pl.when(t > 0)
            def _():
                pltpu.make_async_copy(
                    ob1, out_hbm.at[pl.ds(0, ROWS_STEP)], osem1).wait()

            compute(in1, ob1)
            pltpu.async_copy(ob1, out_hbm.at[pl.ds(row0_of(s0 + 1), ROWS_STEP)],
                             osem1)

        pltpu.make_async_copy(
            emb_hbm.at[pl.ds(0, ROWS_STEP)], in0, isem0).wait()
        pltpu.make_async_copy(
            ob0, out_hbm.at[pl.ds(0, ROWS_STEP)], osem0).wait()
        pltpu.make_async_copy(
            ob1, out_hbm.at[pl.ds(0, ROWS_STEP)], osem1).wait()

    return k(emb, wtile)


def _sc_bagsum(text_flat, table, bias16):
    mesh = plsc.VectorSubcoreMesh(core_axis_name="c", subcore_axis_name="s")

    @functools.partial(
        pl.kernel,
        mesh=mesh,
        out_type=jax.ShapeDtypeStruct((B * NCLS,), jnp.float32),
        scratch_types=[
            pltpu.VMEM((NB * L,), jnp.int32),
            pltpu.VMEM((NB * L, 16), jnp.float32),
            pltpu.VMEM((NB, 16), jnp.float32),
            pltpu.VMEM((NB * NCLS,), jnp.float32),
            pltpu.VMEM((16,), jnp.float32),
            pltpu.SemaphoreType.DMA,
        ],
        compiler_params=pltpu.CompilerParams(
            use_tc_tiling_on_sc=False, needs_layout_passes=False),
    )
    def k(text_hbm, table_hbm, bias_hbm, out_hbm,
          idx_v, rows_v, pacc_v, out_v, bias_v, sem):
        wid = lax.axis_index("s") * NC + lax.axis_index("c")
        pltpu.sync_copy(bias_hbm, bias_v)
        lane = lax.iota(jnp.int32, 16)
        div4 = lane // 4
        mod4 = lane % 4

        @pl.loop(0, NCHUNK)
        def _(g):
            bag0 = wid * BAGS_PER_W + g * NB
            pltpu.sync_copy(text_hbm.at[pl.ds(bag0 * L, NB * L)], idx_v)
            copies = [
                pltpu.async_copy(
                    table_hbm.at[idx_v.at[pl.ds(kk * 128, 128)]],
                    rows_v.at[pl.ds(kk * 128, 128)],
                    sem,
                )
                for kk in range(IDX_ROWS)
            ]
            for c in copies:
                c.wait()
            for i in range(NB):
                def body(k2, acc):
                    r0 = i * L + k2 * 8 + div4
                    a = plsc.load_gather(rows_v, [r0, mod4])
                    b = plsc.load_gather(rows_v, [r0 + 4, mod4])
                    return acc + a + b
                acc = lax.fori_loop(0, L // 8, body,
                                    jnp.zeros((16,), jnp.float32))
                pacc_v[i, :] = acc
            for q in range(NB // 4):
                r = 4 * q + div4
                s = (plsc.load_gather(pacc_v, [r, mod4])
                     + plsc.load_gather(pacc_v, [r, mod4 + 4])
                     + plsc.load_gather(pacc_v, [r, mod4 + 8])
                     + plsc.load_gather(pacc_v, [r, mod4 + 12]))
                out_v[pl.ds(q * 16, 16)] = s + bias_v[...]
            pltpu.sync_copy(out_v, out_hbm.at[pl.ds(bag0 * NCLS, NB * NCLS)])

    return k(text_flat, table, bias16)


def kernel(text, emb_table, fc_w, fc_b):
    text_flat = text.astype(jnp.int32).reshape(B * L)
    w4 = (fc_w.T / jnp.float32(L)).astype(jnp.float32)
    wtile = jnp.tile(w4, (1, 4))
    table = _sc_project(emb_table, wtile)
    bias16 = jnp.tile(fc_b.astype(jnp.float32), 4)
    out_flat = _sc_bagsum(text_flat, table, bias16)
    return out_flat.reshape(B, NCLS)


# speedup vs baseline: 4.2143x; 4.2143x over previous
import functools

import jax
import jax.numpy as jnp
from jax import lax
from jax.experimental import pallas as pl
from jax.experimental.pallas import tpu as pltpu
from jax.experimental.pallas import tpu_sc as plsc

VOCAB = 1000000
D = 32
B = 16384
L = 200
NCLS = 4

NC, NS = 2, 16
NW = NC * NS
BAGS_PER_W = B // NW
NB = 16
NCHUNK = BAGS_PER_W // NB
IDX_ROWS = NB * L // 128


def _tc_project(emb1d, wbig):
    def body(x_ref, w_ref, o_ref):
        x = x_ref[...].reshape(o_ref.shape[0], 256)
        o_ref[...] = jnp.dot(x, w_ref[...], preferred_element_type=jnp.float32)

    rows = VOCAB // 8
    blk = 5000
    return pl.pallas_call(
        body,
        grid=(rows // blk,),
        in_specs=[
            pl.BlockSpec((blk * 256,), lambda i: (i,)),
            pl.BlockSpec((256, 128), lambda i: (0, 0)),
        ],
        out_specs=pl.BlockSpec((blk, 128), lambda i: (i, 0)),
        out_shape=jax.ShapeDtypeStruct((rows, 128), jnp.float32),
    )(emb1d, wbig)


def _sc_bagsum(text_flat, table, bias16):
    mesh = plsc.VectorSubcoreMesh(core_axis_name="c", subcore_axis_name="s")

    @functools.partial(
        pl.kernel,
        mesh=mesh,
        out_type=jax.ShapeDtypeStruct((B * NCLS,), jnp.float32),
        scratch_types=[
            pltpu.VMEM((NB * L,), jnp.int32),
            pltpu.VMEM((NB * L,), jnp.int32),
            pltpu.VMEM((NB * L, 16), jnp.float32),
            pltpu.VMEM((NB * L, 16), jnp.float32),
            pltpu.VMEM((NB, 16), jnp.float32),
            pltpu.VMEM((NB * NCLS,), jnp.float32),
            pltpu.VMEM((16,), jnp.float32),
            pltpu.SemaphoreType.DMA,
            pltpu.SemaphoreType.DMA,
        ],
        compiler_params=pltpu.CompilerParams(
            use_tc_tiling_on_sc=False, needs_layout_passes=False),
    )
    def k(text_hbm, table_hbm, bias_hbm, out_hbm,
          idx0_v, idx1_v, rows0_v, rows1_v, pacc_v, out_v, bias_v,
          sem0, sem1):
        wid = lax.axis_index("s") * NC + lax.axis_index("c")
        pltpu.sync_copy(bias_hbm, bias_v)
        lane = lax.iota(jnp.int32, 16)
        div4 = lane // 4
        mod4 = lane % 4

        def issue(g, idx_v, rows_v, sem):
            bag0 = wid * BAGS_PER_W + g * NB
            pltpu.sync_copy(text_hbm.at[pl.ds(bag0 * L, NB * L)], idx_v)
            for kk in range(IDX_ROWS):
                pltpu.async_copy(
                    table_hbm.at[idx_v.at[pl.ds(kk * 128, 128)]],
                    rows_v.at[pl.ds(kk * 128, 128)],
                    sem,
                )

        def drain(idx_v, rows_v, sem):
            for kk in range(IDX_ROWS):
                pltpu.make_async_copy(
                    table_hbm.at[idx_v.at[pl.ds(kk * 128, 128)]],
                    rows_v.at[pl.ds(kk * 128, 128)],
                    sem,
                ).wait()

        def reduce(g, rows_v):
            bag0 = wid * BAGS_PER_W + g * NB
            for i in range(NB):
                def body(k2, acc):
                    r0 = i * L + k2 * 8 + div4
                    a = plsc.load_gather(rows_v, [r0, mod4])
                    b = plsc.load_gather(rows_v, [r0 + 4, mod4])
                    return acc + a + b
                acc = lax.fori_loop(0, L // 8, body,
                                    jnp.zeros((16,), jnp.float32))
                pacc_v[i, :] = acc
            for q in range(NB // 4):
                r = 4 * q + div4
                s = (plsc.load_gather(pacc_v, [r, mod4])
                     + plsc.load_gather(pacc_v, [r, mod4 + 4])
                     + plsc.load_gather(pacc_v, [r, mod4 + 8])
                     + plsc.load_gather(pacc_v, [r, mod4 + 12]))
                out_v[pl.ds(q * 16, 16)] = s + bias_v[...]
            pltpu.sync_copy(out_v, out_hbm.at[pl.ds(bag0 * NCLS, NB * NCLS)])

        issue(0, idx0_v, rows0_v, sem0)

        @pl.loop(0, NCHUNK // 2)
        def _(t):
            a = 2 * t
            issue(a + 1, idx1_v, rows1_v, sem1)
            drain(idx0_v, rows0_v, sem0)
            reduce(a, rows0_v)

            @pl.when(t < NCHUNK // 2 - 1)
            def _():
                issue(a + 2, idx0_v, rows0_v, sem0)

            drain(idx1_v, rows1_v, sem1)
            reduce(a + 1, rows1_v)

    return k(text_flat, table, bias16)


def kernel(text, emb_table, fc_w, fc_b):
    text_flat = text.astype(jnp.int32).reshape(B * L)
    w16 = jnp.pad((fc_w.T / jnp.float32(L)).astype(jnp.float32),
                  ((0, 0), (0, 16 - NCLS)))
    wbig = jnp.kron(jnp.eye(8, dtype=jnp.float32), w16)
    table = _tc_project(emb_table.reshape(VOCAB * D), wbig).reshape(VOCAB, 16)
    bias16 = jnp.tile(fc_b.astype(jnp.float32), 4)
    out_flat = _sc_bagsum(text_flat, table, bias16)
    return out_flat.reshape(B, NCLS)
